# Initial kernel scaffold; baseline (speedup 1.0000x reference)
import jax
import jax.numpy as jnp
from jax.experimental import pallas as pl


def kernel(x, edge_index, W_l1, W_r1, b1, W_l2, W_r2, b2, W_l3, W_r3, b3):
    raise NotImplementedError("write your pallas kernel here")


# speedup vs baseline: 3.4331x; 3.4331x over previous
import functools

import jax
import jax.numpy as jnp
from jax import lax
from jax.experimental import pallas as pl
from jax.experimental.pallas import tpu as pltpu
from jax.experimental.pallas import tpu_sc as plsc

N = 10000
D = 128
E = 320000
NC = 2
NS = 16
NW = NC * NS
EPW = E // NW
CH = 128
NFULL = EPW // CH
TAIL = EPW - NFULL * CH
ACC_ROWS = 10112
ROWS_PT = ACC_ROWS // NS
OUT_PT = 624
OUT_REM = N - NS * OUT_PT


def _worker(cid, sid):
    return cid * NS + sid


def _load_chunk_idx(idx_hbm, idx_v, base, j):
    pltpu.sync_copy(idx_hbm.at[pl.ds(base + j * CH, CH)], idx_v)


def _prep_tail_idx(src_hbm, dst_hbm, src_v, dst_v, base):
    padi = jnp.full((16,), N, jnp.int32)
    zeroi = jnp.zeros((16,), jnp.int32)
    for k in range(CH // 16):
        src_v[pl.ds(k * 16, 16)] = zeroi
        dst_v[pl.ds(k * 16, 16)] = padi
    off = base + NFULL * CH
    pltpu.sync_copy(src_hbm.at[pl.ds(off, TAIL)], src_v.at[pl.ds(0, TAIL)])
    pltpu.sync_copy(dst_hbm.at[pl.ds(off, TAIL)], dst_v.at[pl.ds(0, TAIL)])


def _copy_out(shared, hbm, cid, sid):
    pltpu.sync_copy(shared.at[pl.ds(sid * OUT_PT, OUT_PT)],
                    hbm.at[cid].at[pl.ds(sid * OUT_PT, OUT_PT)])

    @pl.when(sid == 0)
    def _tail():
        pltpu.sync_copy(shared.at[pl.ds(NS * OUT_PT, OUT_REM)],
                        hbm.at[cid].at[pl.ds(NS * OUT_PT, OUT_REM)])


def _sc_agg_body(x_hbm, src_hbm, dst_hbm, out_hbm,
                 src_v, dst_v, rows_v, acc, sem):
    cid = lax.axis_index("c")
    sid = lax.axis_index("s")
    base = _worker(cid, sid) * EPW
    zero16 = jnp.zeros((16,), jnp.float32)

    def zfill(i, _):
        rows_v[i // (D // 16), pl.ds((i % (D // 16)) * 16, 16)] = zero16
        return 0
    lax.fori_loop(0, CH * (D // 16), zfill, 0)
    base_r = sid * ROWS_PT
    for t in range(ROWS_PT // CH):
        pltpu.sync_copy(rows_v, acc.at[pl.ds(base_r + t * CH, CH)])
    zrem = ROWS_PT - (ROWS_PT // CH) * CH
    if zrem:
        pltpu.sync_copy(rows_v.at[pl.ds(0, zrem)],
                        acc.at[pl.ds(base_r + (ROWS_PT // CH) * CH, zrem)])

    plsc.subcore_barrier()

    def chunk(j, _):
        _load_chunk_idx(src_hbm, src_v, base, j)
        _load_chunk_idx(dst_hbm, dst_v, base, j)
        pltpu.async_copy(x_hbm.at[src_v], rows_v, sem).wait()
        pltpu.sync_copy(rows_v, acc.at[dst_v], add=True)
        return 0
    lax.fori_loop(0, NFULL, chunk, 0)

    _prep_tail_idx(src_hbm, dst_hbm, src_v, dst_v, base)
    pltpu.async_copy(x_hbm.at[src_v], rows_v, sem).wait()
    pltpu.sync_copy(rows_v, acc.at[dst_v], add=True)

    plsc.subcore_barrier()
    _copy_out(acc, out_hbm, cid, sid)


def _sc_deg_body(src_hbm, dst_hbm, deg_hbm,
                 src_v, dst_v, ones_v, zdeg, degacc):
    cid = lax.axis_index("c")
    sid = lax.axis_index("s")
    base = _worker(cid, sid) * EPW
    zero16 = jnp.zeros((16,), jnp.float32)
    one16 = jnp.ones((16,), jnp.float32)

    def zdfill(i, _):
        zdeg[i, :] = zero16
        return 0
    lax.fori_loop(0, zdeg.shape[0], zdfill, 0)

    def ofill(i, _):
        ones_v[i, :] = one16
        return 0
    lax.fori_loop(0, CH, ofill, 0)

    ZD = zdeg.shape[0]
    base_r = sid * ROWS_PT
    for t in range(ROWS_PT // ZD):
        pltpu.sync_copy(zdeg, degacc.at[pl.ds(base_r + t * ZD, ZD)])
    drem = ROWS_PT - (ROWS_PT // ZD) * ZD
    if drem:
        pltpu.sync_copy(zdeg.at[pl.ds(0, drem)],
                        degacc.at[pl.ds(base_r + (ROWS_PT // ZD) * ZD, drem)])

    plsc.subcore_barrier()

    def chunk(j, _):
        _load_chunk_idx(dst_hbm, dst_v, base, j)
        pltpu.sync_copy(ones_v, degacc.at[dst_v], add=True)
        return 0
    lax.fori_loop(0, NFULL, chunk, 0)

    _prep_tail_idx(src_hbm, dst_hbm, src_v, dst_v, base)
    pltpu.sync_copy(ones_v, degacc.at[dst_v], add=True)

    plsc.subcore_barrier()
    _copy_out(degacc, deg_hbm, cid, sid)


def _sc_mesh():
    return plsc.VectorSubcoreMesh(core_axis_name="c", subcore_axis_name="s",
                                  num_cores=NC, num_subcores=NS)


_sc_agg = pl.kernel(
    _sc_agg_body,
    out_type=jax.ShapeDtypeStruct((NC, N, D), jnp.float32),
    mesh=_sc_mesh(),
    scratch_types=[
        pltpu.VMEM((CH,), jnp.int32),
        pltpu.VMEM((CH,), jnp.int32),
        pltpu.VMEM((CH, D), jnp.float32),
        pltpu.VMEM_SHARED((ACC_ROWS, D), jnp.float32),
        pltpu.SemaphoreType.DMA,
    ],
    name="sc_agg",
)

_sc_deg = pl.kernel(
    _sc_deg_body,
    out_type=jax.ShapeDtypeStruct((NC, N, 16), jnp.float32),
    mesh=_sc_mesh(),
    scratch_types=[
        pltpu.VMEM((CH,), jnp.int32),
        pltpu.VMEM((CH,), jnp.int32),
        pltpu.VMEM((CH, 16), jnp.float32),
        pltpu.VMEM((64, 16), jnp.float32),
        pltpu.VMEM_SHARED((ACC_ROWS, 16), jnp.float32),
    ],
    name="sc_deg",
)


def _tc_combine_body(relu, p0, p1, d0, d1, xr, wl, wr, bb, o):
    deg = d0[:, 0:1] + d1[:, 0:1]
    inv = 1.0 / jnp.maximum(deg, 1.0)
    mean = (p0[...] + p1[...]) * inv
    acc = jnp.dot(mean, wl[...], preferred_element_type=jnp.float32)
    acc = acc + jnp.dot(xr[...], wr[...], preferred_element_type=jnp.float32)
    acc = acc + bb[...]
    o[...] = jnp.maximum(acc, 0.0) if relu else acc


_TC_R = 1000


def _make_tc_combine(relu):
    row = lambda i: (i, 0)
    fixed = lambda i: (0, 0)
    return pl.pallas_call(
        functools.partial(_tc_combine_body, relu),
        grid=(N // _TC_R,),
        in_specs=[
            pl.BlockSpec((_TC_R, D), row),
            pl.BlockSpec((_TC_R, D), row),
            pl.BlockSpec((_TC_R, 16), row),
            pl.BlockSpec((_TC_R, 16), row),
            pl.BlockSpec((_TC_R, D), row),
            pl.BlockSpec((D, D), fixed),
            pl.BlockSpec((D, D), fixed),
            pl.BlockSpec((1, D), fixed),
        ],
        out_specs=pl.BlockSpec((_TC_R, D), row),
        out_shape=jax.ShapeDtypeStruct((N, D), jnp.float32),
    )


_tc_relu = _make_tc_combine(True)
_tc_plain = _make_tc_combine(False)


def kernel(x, edge_index, W_l1, W_r1, b1, W_l2, W_r2, b2, W_l3, W_r3, b3):
    src = edge_index[0]
    dst = edge_index[1]
    ones = jnp.ones((N, D), jnp.float32)
    deg = _sc_agg(ones, src, dst)
    d0, d1 = deg[0][:, :16], deg[1][:, :16]
    part1 = _sc_agg(x, src, dst)
    h1 = _tc_relu(part1[0], part1[1], d0, d1, x,
                  W_l1.T, W_r1.T, b1.reshape(1, D))
    part2 = _sc_agg(h1, src, dst)
    h2 = _tc_relu(part2[0], part2[1], d0, d1, h1,
                  W_l2.T, W_r2.T, b2.reshape(1, D))
    part3 = _sc_agg(h2, src, dst)
    h3 = _tc_plain(part3[0], part3[1], d0, d1, h2,
                   W_l3.T, W_r3.T, b3.reshape(1, D))
    return h3
